# baseline (speedup 1.0000x reference)
import jax
import jax.numpy as jnp
from jax import lax
from jax.experimental import pallas as pl

EMB_D = 64
USR_D = 16
NEXP = 64
PACK = 8
ROWS = 512


def _top2(logits):
    row = lax.broadcasted_iota(jnp.int32, (NEXP, NEXP), 0)
    col = lax.broadcasted_iota(jnp.int32, (NEXP, NEXP), 1)
    lt = (row <= col).astype(jnp.float32)
    m1 = jnp.max(logits, axis=1, keepdims=True)
    eq1 = logits == m1
    cs1 = jnp.dot(eq1.astype(jnp.float32), lt,
                  preferred_element_type=jnp.float32)
    mask1 = eq1 & (cs1 == 1.0)
    l2 = jnp.where(mask1, -jnp.inf, logits)
    m2 = jnp.max(l2, axis=1, keepdims=True)
    eq2 = l2 == m2
    cs2 = jnp.dot(eq2.astype(jnp.float32), lt,
                  preferred_element_type=jnp.float32)
    mask2 = eq2 & (cs2 == 1.0)
    e = jnp.exp(m2 - m1)
    p1 = 1.0 / (1.0 + e)
    p2 = 1.0 - p1
    return jnp.where(mask1, p1, jnp.where(mask2, p2, 0.0))


def _gate_body(h_ref, u_ref, wg_ref, bg_ref, wb_ref, bb_ref, wl_ref,
               bl_ref, out_ref):
    wg = wg_ref[...]
    bg = bg_ref[...]
    wb = wb_ref[...]
    bb = bb_ref[...]
    wl = wl_ref[...]
    bl = bl_ref[...]
    for k in range(PACK):
        uk = u_ref[:, k * USR_D:(k + 1) * USR_D]
        hk = h_ref[:, k * EMB_D:(k + 1) * EMB_D]
        gamma = jnp.dot(uk, wg, preferred_element_type=jnp.float32) + bg
        beta = jnp.dot(uk, wb, preferred_element_type=jnp.float32) + bb
        h_t = hk * (1.0 + gamma) + beta
        logits = jnp.dot(h_t, wl, preferred_element_type=jnp.float32) + bl
        out_ref[:, k * NEXP:(k + 1) * NEXP] = _top2(logits)


def kernel(h, u, Wg, bg, Wb, bb, Wl, bl):
    n = h.shape[0]
    h8 = h.reshape(n // PACK, PACK * EMB_D)
    u8 = u.reshape(n // PACK, PACK * USR_D)
    grid = (n // PACK // ROWS,)
    w8 = pl.pallas_call(
        _gate_body,
        grid=grid,
        in_specs=[
            pl.BlockSpec((ROWS, PACK * EMB_D), lambda i: (i, 0)),
            pl.BlockSpec((ROWS, PACK * USR_D), lambda i: (i, 0)),
            pl.BlockSpec((USR_D, EMB_D), lambda i: (0, 0)),
            pl.BlockSpec((1, EMB_D), lambda i: (0, 0)),
            pl.BlockSpec((USR_D, EMB_D), lambda i: (0, 0)),
            pl.BlockSpec((1, EMB_D), lambda i: (0, 0)),
            pl.BlockSpec((EMB_D, NEXP), lambda i: (0, 0)),
            pl.BlockSpec((1, NEXP), lambda i: (0, 0)),
        ],
        out_specs=pl.BlockSpec((ROWS, PACK * NEXP), lambda i: (i, 0)),
        out_shape=jax.ShapeDtypeStruct((n // PACK, PACK * NEXP), jnp.float32),
    )(h8, u8, Wg.T, bg[None, :], Wb.T, bb[None, :], Wl.T, bl[None, :])
    return w8.reshape(n, NEXP)


# speedup vs baseline: 2.0341x; 2.0341x over previous
import jax
import jax.numpy as jnp
from jax import lax
from jax.experimental import pallas as pl

EMB_D = 64
USR_D = 16
NEXP = 64
BLK = 4096


def _gate_body(h_ref, u_ref, wg_ref, bg_ref, wb_ref, bb_ref, wl_ref,
               bl_ref, out_ref):
    u = u_ref[...]
    h = h_ref[...]
    gamma = jnp.dot(u, wg_ref[...], preferred_element_type=jnp.float32)
    gamma = gamma + bg_ref[...]
    beta = jnp.dot(u, wb_ref[...], preferred_element_type=jnp.float32)
    beta = beta + bb_ref[...]
    h_t = h * (1.0 + gamma) + beta
    logits = jnp.dot(h_t, wl_ref[...], preferred_element_type=jnp.float32)
    logits = logits + bl_ref[...]

    row = lax.broadcasted_iota(jnp.int32, (NEXP, NEXP), 0)
    col = lax.broadcasted_iota(jnp.int32, (NEXP, NEXP), 1)
    lt = (row <= col).astype(jnp.float32)

    m1 = jnp.max(logits, axis=1, keepdims=True)
    eq1 = logits == m1
    cs1 = jnp.dot(eq1.astype(jnp.float32), lt,
                  preferred_element_type=jnp.float32)
    mask1 = eq1 & (cs1 == 1.0)
    l2 = jnp.where(mask1, -jnp.inf, logits)
    m2 = jnp.max(l2, axis=1, keepdims=True)
    eq2 = l2 == m2
    cs2 = jnp.dot(eq2.astype(jnp.float32), lt,
                  preferred_element_type=jnp.float32)
    mask2 = eq2 & (cs2 == 1.0)

    e = jnp.exp(m2 - m1)
    denom = 1.0 + e
    p1 = 1.0 / denom
    p2 = e / denom
    out_ref[...] = jnp.where(mask1, p1, jnp.where(mask2, p2, 0.0))


def kernel(h, u, Wg, bg, Wb, bb, Wl, bl):
    n = h.shape[0]
    grid = (n // BLK,)
    return pl.pallas_call(
        _gate_body,
        grid=grid,
        in_specs=[
            pl.BlockSpec((BLK, EMB_D), lambda i: (i, 0)),
            pl.BlockSpec((BLK, USR_D), lambda i: (i, 0)),
            pl.BlockSpec((USR_D, EMB_D), lambda i: (0, 0)),
            pl.BlockSpec((1, EMB_D), lambda i: (0, 0)),
            pl.BlockSpec((USR_D, EMB_D), lambda i: (0, 0)),
            pl.BlockSpec((1, EMB_D), lambda i: (0, 0)),
            pl.BlockSpec((EMB_D, NEXP), lambda i: (0, 0)),
            pl.BlockSpec((1, NEXP), lambda i: (0, 0)),
        ],
        out_specs=pl.BlockSpec((BLK, NEXP), lambda i: (i, 0)),
        out_shape=jax.ShapeDtypeStruct((n, NEXP), jnp.float32),
    )(h, u, Wg.T, bg[None, :], Wb.T, bb[None, :], Wl.T, bl[None, :])
